# baseline (speedup 1.0000x reference)
import jax
import jax.numpy as jnp
from jax import lax
from jax.experimental import pallas as pl
from jax.experimental.pallas import tpu as pltpu

N_TOK = 8192
D_IN = 1024
D_HID = 4096
TOPK_WIDTH = 64
BT = 256
NSPLIT = 1
NB = N_TOK // BT


def _count_gt(v16, m16):
    c = (v16 > m16).astype(jnp.int16)
    w = D_HID // 2
    while w >= 128:
        c = c[:, :w] + c[:, w:]
        w //= 2
    return jnp.sum(c.astype(jnp.int32), axis=1, keepdims=True)


def _bisect_multi(vs, tgts, lo0, hi0, iters):
    los, his = list(lo0), list(hi0)
    chis = [jnp.zeros_like(lo) for lo in lo0]
    for _ in range(iters):
        mids = [lo + (hi - lo) // 2 for lo, hi in zip(los, his)]
        cnts = [_count_gt(v, m.astype(jnp.int16))
                for v, m in zip(vs, mids)]
        ges = [c >= t for c, t in zip(cnts, tgts)]
        los = [jnp.where(g, m, lo) for g, m, lo in zip(ges, mids, los)]
        his = [jnp.where(g, hi, m) for g, m, hi in zip(ges, mids, his)]
        chis = [jnp.where(g, ch, c) for g, ch, c in zip(ges, chis, cnts)]
    return los, chis


def _topk_mask(h, kk):
    hb = lax.bitcast_convert_type(h, jnp.int32)
    rows = h.shape[0]
    sub = rows // NSPLIT
    hbs = [hb[i * sub:(i + 1) * sub] for i in range(NSPLIT)]
    ones = jnp.ones((sub, 1), jnp.int32)

    hi16s = [(b >> 16).astype(jnp.int16) for b in hbs]
    t_his, c_aboves = _bisect_multi(hi16s, [kk] * NSPLIT, [-1 * ones] * NSPLIT,
                                    [0x7F80 * ones] * NSPLIT, 15)

    bkt16s = [(t + 1).astype(jnp.int16) for t in t_his]
    js = [kk - c for c in c_aboves]

    eqs = [v == b for v, b in zip(hi16s, bkt16s)]
    lss = [jnp.where(e, ((b & 0xFFFF) - 0x8000).astype(jnp.int16),
                     jnp.int16(-0x8000)) for e, b in zip(eqs, hbs)]
    t_los, _ = _bisect_multi(lss, js, [-0x8001 * ones] * NSPLIT,
                             [0x7FFF * ones] * NSPLIT, 16)

    masks = [(v > b) | (e & ((ls > t.astype(jnp.int16)) | (t == -0x8001)))
             for v, b, e, ls, t in zip(hi16s, bkt16s, eqs, lss, t_los)]
    mask = jnp.concatenate(masks, axis=0)
    return jnp.where(mask, h, 0.0)


def _fused_body(k_ref, x_ref, we_ref, be_ref, wd_ref, recon_ref, h_ref,
                h_scr):
    i = pl.program_id(0)
    cur = jax.lax.rem(i, 2)
    prv = jax.lax.rem(i + 1, 2)

    h_pre = lax.dot_general(
        x_ref[...], we_ref[...],
        dimension_numbers=(((1,), (1,)), ((), ())),
        preferred_element_type=jnp.float32,
    )
    h_scr[cur] = jnp.maximum(h_pre + be_ref[...], 0.0)

    kk = jnp.minimum(k_ref[0], TOPK_WIDTH)
    h = h_scr[prv]
    h_m = _topk_mask(h, kk)
    h_ref[...] = h_m
    recon_ref[...] = lax.dot_general(
        h_m, wd_ref[...],
        dimension_numbers=(((1,), (1,)), ((), ())),
        preferred_element_type=jnp.float32,
    )


@jax.jit
def _run(x, kk, W_enc, b_enc, W_dec):
    grid = (NB + 1,)
    return pl.pallas_call(
        _fused_body,
        grid=grid,
        in_specs=[
            pl.BlockSpec(memory_space=pltpu.SMEM),
            pl.BlockSpec((BT, D_IN), lambda i: (jnp.minimum(i, NB - 1), 0)),
            pl.BlockSpec((D_HID, D_IN), lambda i: (0, 0)),
            pl.BlockSpec((1, D_HID), lambda i: (0, 0)),
            pl.BlockSpec((D_IN, D_HID), lambda i: (0, 0)),
        ],
        out_specs=[
            pl.BlockSpec((BT, D_IN), lambda i: (jnp.maximum(i - 1, 0), 0)),
            pl.BlockSpec((BT, D_HID), lambda i: (jnp.maximum(i - 1, 0), 0)),
        ],
        out_shape=[
            jax.ShapeDtypeStruct((N_TOK, D_IN), jnp.float32),
            jax.ShapeDtypeStruct((N_TOK, D_HID), jnp.float32),
        ],
        scratch_shapes=[pltpu.VMEM((2, BT, D_HID), jnp.float32)],
    )(kk, x, W_enc, b_enc.reshape(1, D_HID), W_dec)


def kernel(x, k, W_enc, b_enc, W_dec):
    kk = jnp.asarray(k, jnp.int32).reshape(1)
    recon, h = _run(x, kk, W_enc, b_enc, W_dec)
    return (recon, h)


# speedup vs baseline: 1.0209x; 1.0209x over previous
import jax
import jax.numpy as jnp
from jax import lax
from jax.experimental import pallas as pl
from jax.experimental.pallas import tpu as pltpu

N_TOK = 8192
D_IN = 1024
D_HID = 4096
TOPK_WIDTH = 64
BT = 256
NSPLIT = 1
NB = N_TOK // BT


def _count_gt(v16, m16):
    c = (v16 > m16).astype(jnp.int16)
    w = D_HID // 4
    while w >= 128:
        c = c[:, :w] + c[:, w:2 * w] + c[:, 2 * w:3 * w] + c[:, 3 * w:]
        w //= 4
    if w * 2 == 128:
        c = c[:, :128] + c[:, 128:]
    return jnp.sum(c.astype(jnp.int32), axis=1, keepdims=True)


def _bisect_multi(vs, tgts, lo0, hi0, iters):
    los, his = list(lo0), list(hi0)
    chis = [jnp.zeros_like(lo) for lo in lo0]
    for _ in range(iters):
        mids = [lo + (hi - lo) // 2 for lo, hi in zip(los, his)]
        cnts = [_count_gt(v, m.astype(jnp.int16))
                for v, m in zip(vs, mids)]
        ges = [c >= t for c, t in zip(cnts, tgts)]
        los = [jnp.where(g, m, lo) for g, m, lo in zip(ges, mids, los)]
        his = [jnp.where(g, hi, m) for g, m, hi in zip(ges, mids, his)]
        chis = [jnp.where(g, ch, c) for g, ch, c in zip(ges, chis, cnts)]
    return los, chis


def _topk_mask(h, kk):
    hb = lax.bitcast_convert_type(h, jnp.int32)
    rows = h.shape[0]
    sub = rows // NSPLIT
    hbs = [hb[i * sub:(i + 1) * sub] for i in range(NSPLIT)]
    ones = jnp.ones((sub, 1), jnp.int32)

    hi16s = [(b >> 16).astype(jnp.int16) for b in hbs]
    t_his, c_aboves = _bisect_multi(hi16s, [kk] * NSPLIT, [-1 * ones] * NSPLIT,
                                    [0x7F80 * ones] * NSPLIT, 15)

    bkt16s = [(t + 1).astype(jnp.int16) for t in t_his]
    js = [kk - c for c in c_aboves]

    eqs = [v == b for v, b in zip(hi16s, bkt16s)]
    lss = [jnp.where(e, (b ^ 0x8000).astype(jnp.int16), jnp.int16(-0x8000))
           for e, b in zip(eqs, hbs)]
    t_los, _ = _bisect_multi(lss, js, [-0x8001 * ones] * NSPLIT,
                             [0x7FFF * ones] * NSPLIT, 16)

    masks = [(v > b) | (e & ((ls > t.astype(jnp.int16)) | (t == -0x8001)))
             for v, b, e, ls, t in zip(hi16s, bkt16s, eqs, lss, t_los)]
    mask = jnp.concatenate(masks, axis=0)
    return jnp.where(mask, h, 0.0)


def _fused_body(k_ref, x_ref, we_ref, be_ref, wd_ref, recon_ref, h_ref,
                h_scr):
    i = pl.program_id(0)
    cur = jax.lax.rem(i, 2)
    prv = jax.lax.rem(i + 1, 2)

    h_pre = lax.dot_general(
        x_ref[...], we_ref[...],
        dimension_numbers=(((1,), (1,)), ((), ())),
        preferred_element_type=jnp.float32,
    )
    h_scr[cur] = jnp.maximum(h_pre + be_ref[...], 0.0)

    kk = jnp.minimum(k_ref[0], TOPK_WIDTH)
    h = h_scr[prv]
    h_m = _topk_mask(h, kk)
    h_ref[...] = h_m
    recon_ref[...] = lax.dot_general(
        h_m, wd_ref[...],
        dimension_numbers=(((1,), (1,)), ((), ())),
        preferred_element_type=jnp.float32,
    )


@jax.jit
def _run(x, kk, W_enc, b_enc, W_dec):
    grid = (NB + 1,)
    return pl.pallas_call(
        _fused_body,
        grid=grid,
        in_specs=[
            pl.BlockSpec(memory_space=pltpu.SMEM),
            pl.BlockSpec((BT, D_IN), lambda i: (jnp.minimum(i, NB - 1), 0)),
            pl.BlockSpec((D_HID, D_IN), lambda i: (0, 0)),
            pl.BlockSpec((1, D_HID), lambda i: (0, 0)),
            pl.BlockSpec((D_IN, D_HID), lambda i: (0, 0)),
        ],
        out_specs=[
            pl.BlockSpec((BT, D_IN), lambda i: (jnp.maximum(i - 1, 0), 0)),
            pl.BlockSpec((BT, D_HID), lambda i: (jnp.maximum(i - 1, 0), 0)),
        ],
        out_shape=[
            jax.ShapeDtypeStruct((N_TOK, D_IN), jnp.float32),
            jax.ShapeDtypeStruct((N_TOK, D_HID), jnp.float32),
        ],
        scratch_shapes=[pltpu.VMEM((2, BT, D_HID), jnp.float32)],
    )(kk, x, W_enc, b_enc.reshape(1, D_HID), W_dec)


def kernel(x, k, W_enc, b_enc, W_dec):
    kk = jnp.asarray(k, jnp.int32).reshape(1)
    recon, h = _run(x, kk, W_enc, b_enc, W_dec)
    return (recon, h)
